# baseline (speedup 1.0000x reference)
import jax
import jax.numpy as jnp
from jax.experimental import pallas as pl
from jax.experimental.pallas import tpu as pltpu

_BM = 200


def _xw_kernel(x_ref, w_ref, o_ref):
    o_ref[...] = jax.lax.dot_general(
        x_ref[...].astype(jnp.bfloat16),
        w_ref[...],
        (((1,), (1,)), ((), ())),
        preferred_element_type=jnp.float32,
    ).astype(jnp.bfloat16)


def _agg1_kernel(a_ref, h_ref, w2_ref, o_ref):
    acc = jnp.dot(
        a_ref[...].astype(jnp.bfloat16),
        h_ref[...],
        preferred_element_type=jnp.float32,
    )
    acc = jnp.maximum(acc, 0.0).astype(jnp.bfloat16)
    o_ref[...] = jax.lax.dot_general(
        acc,
        w2_ref[...],
        (((1,), (1,)), ((), ())),
        preferred_element_type=jnp.float32,
    ).astype(jnp.bfloat16)


def _agg2_kernel(a_ref, h_ref, o_ref):
    acc = jnp.dot(
        a_ref[...].astype(jnp.bfloat16),
        h_ref[...],
        preferred_element_type=jnp.float32,
    )
    m = jnp.max(acc, axis=1, keepdims=True)
    lse = jnp.log(jnp.sum(jnp.exp(acc - m), axis=1, keepdims=True))
    o_ref[...] = acc - m - lse


def kernel(x, adjacency, W1, W2):
    n, d_in = x.shape
    d_h = W1.shape[0]
    d_out = W2.shape[0]
    bm = min(_BM, n)
    grid = (n // bm,)

    w1_b = W1.astype(jnp.bfloat16)
    w2_b = W2.astype(jnp.bfloat16)

    xw1 = pl.pallas_call(
        _xw_kernel,
        grid=grid,
        in_specs=[
            pl.BlockSpec((bm, d_in), lambda i: (i, 0)),
            pl.BlockSpec((d_h, d_in), lambda i: (0, 0)),
        ],
        out_specs=pl.BlockSpec((bm, d_h), lambda i: (i, 0)),
        out_shape=jax.ShapeDtypeStruct((n, d_h), jnp.bfloat16),
        compiler_params=pltpu.CompilerParams(
            dimension_semantics=("arbitrary",),
        ),
    )(x, w1_b)

    hw2 = pl.pallas_call(
        _agg1_kernel,
        grid=grid,
        in_specs=[
            pl.BlockSpec((bm, n), lambda i: (i, 0)),
            pl.BlockSpec((n, d_h), lambda i: (0, 0)),
            pl.BlockSpec((d_out, d_h), lambda i: (0, 0)),
        ],
        out_specs=pl.BlockSpec((bm, d_out), lambda i: (i, 0)),
        out_shape=jax.ShapeDtypeStruct((n, d_out), jnp.bfloat16),
        compiler_params=pltpu.CompilerParams(
            dimension_semantics=("arbitrary",),
        ),
    )(adjacency, xw1, w2_b)

    out = pl.pallas_call(
        _agg2_kernel,
        grid=grid,
        in_specs=[
            pl.BlockSpec((bm, n), lambda i: (i, 0)),
            pl.BlockSpec((n, d_out), lambda i: (0, 0)),
        ],
        out_specs=pl.BlockSpec((bm, d_out), lambda i: (i, 0)),
        out_shape=jax.ShapeDtypeStruct((n, d_out), jnp.float32),
        compiler_params=pltpu.CompilerParams(
            dimension_semantics=("arbitrary",),
        ),
    )(adjacency, hw2)

    return out


# speedup vs baseline: 1.1621x; 1.1621x over previous
import jax
import jax.numpy as jnp
from jax.experimental import pallas as pl
from jax.experimental.pallas import tpu as pltpu

_BM = 400


def _xw_kernel(x_ref, w_ref, o_ref):
    o_ref[...] = jax.lax.dot_general(
        x_ref[...].astype(jnp.bfloat16),
        w_ref[...],
        (((1,), (1,)), ((), ())),
        preferred_element_type=jnp.float32,
    ).astype(jnp.bfloat16)


def _agg1_kernel(a_ref, h_ref, w2_ref, o_ref):
    acc = jnp.dot(
        a_ref[...].astype(jnp.bfloat16),
        h_ref[...],
        preferred_element_type=jnp.float32,
    )
    acc = jnp.maximum(acc, 0.0).astype(jnp.bfloat16)
    o_ref[...] = jax.lax.dot_general(
        acc,
        w2_ref[...],
        (((1,), (1,)), ((), ())),
        preferred_element_type=jnp.float32,
    ).astype(jnp.bfloat16)


def _agg2_kernel(a_ref, h_ref, o_ref):
    acc = jnp.dot(
        a_ref[...].astype(jnp.bfloat16),
        h_ref[...],
        preferred_element_type=jnp.float32,
    )
    m = jnp.max(acc, axis=1, keepdims=True)
    lse = jnp.log(jnp.sum(jnp.exp(acc - m), axis=1, keepdims=True))
    o_ref[...] = acc - m - lse


def kernel(x, adjacency, W1, W2):
    n, d_in = x.shape
    d_h = W1.shape[0]
    d_out = W2.shape[0]
    bm = min(_BM, n)
    grid = (n // bm,)

    w1_b = W1.astype(jnp.bfloat16)
    w2_b = W2.astype(jnp.bfloat16)

    xw1 = pl.pallas_call(
        _xw_kernel,
        grid=grid,
        in_specs=[
            pl.BlockSpec((bm, d_in), lambda i: (i, 0)),
            pl.BlockSpec((d_h, d_in), lambda i: (0, 0)),
        ],
        out_specs=pl.BlockSpec((bm, d_h), lambda i: (i, 0)),
        out_shape=jax.ShapeDtypeStruct((n, d_h), jnp.bfloat16),
        compiler_params=pltpu.CompilerParams(
            dimension_semantics=("arbitrary",),
        ),
    )(x, w1_b)

    hw2 = pl.pallas_call(
        _agg1_kernel,
        grid=grid,
        in_specs=[
            pl.BlockSpec((bm, n), lambda i: (i, 0)),
            pl.BlockSpec((n, d_h), lambda i: (0, 0)),
            pl.BlockSpec((d_out, d_h), lambda i: (0, 0)),
        ],
        out_specs=pl.BlockSpec((bm, d_out), lambda i: (i, 0)),
        out_shape=jax.ShapeDtypeStruct((n, d_out), jnp.bfloat16),
        compiler_params=pltpu.CompilerParams(
            dimension_semantics=("arbitrary",),
        ),
    )(adjacency, xw1, w2_b)

    out = pl.pallas_call(
        _agg2_kernel,
        grid=grid,
        in_specs=[
            pl.BlockSpec((bm, n), lambda i: (i, 0)),
            pl.BlockSpec((n, d_out), lambda i: (0, 0)),
        ],
        out_specs=pl.BlockSpec((bm, d_out), lambda i: (i, 0)),
        out_shape=jax.ShapeDtypeStruct((n, d_out), jnp.float32),
        compiler_params=pltpu.CompilerParams(
            dimension_semantics=("arbitrary",),
        ),
    )(adjacency, hw2)

    return out


# speedup vs baseline: 1.2397x; 1.0668x over previous
import jax
import jax.numpy as jnp
from jax.experimental import pallas as pl
from jax.experimental.pallas import tpu as pltpu

_BM = 400


def _fused_kernel(x_ref, a_ref, w1_ref, w2_ref, o_ref, xw1_s, hw2_s):
    p = pl.program_id(0)
    i = pl.program_id(1)
    bm = x_ref.shape[0]

    @pl.when(p == 0)
    def _phase0():
        xw1_s[pl.ds(i * bm, bm), :] = jax.lax.dot_general(
            x_ref[...].astype(jnp.bfloat16),
            w1_ref[...],
            (((1,), (1,)), ((), ())),
            preferred_element_type=jnp.float32,
        ).astype(jnp.bfloat16)

    @pl.when(p == 1)
    def _phase1():
        acc = jnp.dot(
            a_ref[...].astype(jnp.bfloat16),
            xw1_s[...],
            preferred_element_type=jnp.float32,
        )
        acc = jnp.maximum(acc, 0.0).astype(jnp.bfloat16)
        hw2_s[pl.ds(i * bm, bm), :] = jax.lax.dot_general(
            acc,
            w2_ref[...],
            (((1,), (1,)), ((), ())),
            preferred_element_type=jnp.float32,
        ).astype(jnp.bfloat16)

    @pl.when(p == 2)
    def _phase2():
        acc = jnp.dot(
            a_ref[...].astype(jnp.bfloat16),
            hw2_s[...],
            preferred_element_type=jnp.float32,
        )
        m = jnp.max(acc, axis=1, keepdims=True)
        lse = jnp.log(jnp.sum(jnp.exp(acc - m), axis=1, keepdims=True))
        o_ref[...] = acc - m - lse


def kernel(x, adjacency, W1, W2):
    n, d_in = x.shape
    d_h = W1.shape[0]
    d_out = W2.shape[0]
    bm = min(_BM, n)
    num_i = n // bm
    last = num_i - 1

    w1_b = W1.astype(jnp.bfloat16)
    w2_b = W2.astype(jnp.bfloat16)

    def x_map(p, i):
        return (jnp.where(p == 0, i, last), 0)

    def a_map(p, i):
        return (jnp.where(p == 0, 0, jnp.where(p == 1, i, last - i)), 0)

    def o_map(p, i):
        return (jnp.where(p == 2, last - i, last), 0)

    out = pl.pallas_call(
        _fused_kernel,
        grid=(3, num_i),
        in_specs=[
            pl.BlockSpec((bm, d_in), x_map),
            pl.BlockSpec((bm, n), a_map),
            pl.BlockSpec((d_h, d_in), lambda p, i: (0, 0)),
            pl.BlockSpec((d_out, d_h), lambda p, i: (0, 0)),
        ],
        out_specs=pl.BlockSpec((bm, d_out), o_map),
        out_shape=jax.ShapeDtypeStruct((n, d_out), jnp.float32),
        scratch_shapes=[
            pltpu.VMEM((n, d_h), jnp.bfloat16),
            pltpu.VMEM((n, d_out), jnp.bfloat16),
        ],
        compiler_params=pltpu.CompilerParams(
            dimension_semantics=("arbitrary", "arbitrary"),
        ),
    )(x, adjacency, w1_b, w2_b)

    return out
